# baseline (speedup 1.0000x reference)
import functools

import jax
import jax.numpy as jnp
from jax import lax
from jax.experimental import pallas as pl
from jax.experimental.pallas import tpu as pltpu
from jax.experimental.pallas import tpu_sc as plsc

NC, NS = 2, 16
NW = NC * NS


def _sc_gather(table, idx, dim, ch=128, nbuf=5, tc_tiling=True):
    n = idx.shape[0]
    per_w = n // NW
    n_ch = per_w // ch
    n_grp = n_ch // nbuf
    assert per_w % ch == 0 and n_ch % nbuf == 0
    spmem = table.shape[0] * dim * 4 <= 6 * 2**20
    mesh = plsc.VectorSubcoreMesh(
        core_axis_name="c", subcore_axis_name="s", num_cores=NC, num_subcores=NS
    )

    @functools.partial(
        pl.kernel,
        out_type=jax.ShapeDtypeStruct((n, dim), jnp.float32),
        mesh=mesh,
        compiler_params=pltpu.CompilerParams(use_tc_tiling_on_sc=tc_tiling),
        scratch_types=[
            pltpu.VMEM((per_w,), jnp.int32),
            [pltpu.VMEM((ch, dim), jnp.float32)] * nbuf,
            [pltpu.SemaphoreType.DMA] * nbuf,
            [pltpu.SemaphoreType.DMA] * nbuf,
            [pltpu.VMEM_SHARED(table.shape, jnp.float32)] * (1 if spmem else 0),
            [pltpu.SemaphoreType.DMA] * (1 if spmem else 0),
        ],
    )
    def k(table_hbm, idx_hbm, out_hbm, idx_v, bufs, sem_g, sem_w, shared, sem_t):
        wid = lax.axis_index("s") * NC + lax.axis_index("c")
        w_base = wid * per_w

        if spmem:

            @pl.when(lax.axis_index("s") == 0)
            def _():
                pltpu.make_async_copy(table_hbm, shared[0], sem_t[0]).start()
                pltpu.make_async_copy(table_hbm, shared[0], sem_t[0]).wait()

            plsc.subcore_barrier()
            t_src = shared[0]
        else:
            t_src = table_hbm

        pltpu.sync_copy(idx_hbm.at[pl.ds(w_base, per_w)], idx_v)

        def start_gather(s, j):
            pltpu.make_async_copy(
                t_src.at[idx_v.at[pl.ds(j * ch, ch)]], bufs[s], sem_g[s]
            ).start()

        def start_write(s, j):
            pltpu.make_async_copy(
                bufs[s], out_hbm.at[pl.ds(w_base + j * ch, ch)], sem_w[s]
            ).start()

        def wait_gather(s):
            pltpu.make_async_copy(
                t_src.at[idx_v.at[pl.ds(0, ch)]], bufs[s], sem_g[s]
            ).wait()

        def wait_write(s):
            pltpu.make_async_copy(bufs[s], out_hbm.at[pl.ds(0, ch)], sem_w[s]).wait()

        for s in range(nbuf):
            start_gather(s, s)

        @pl.loop(0, n_grp - 1)
        def _(grp):
            g0 = grp * nbuf
            for s in range(nbuf):
                wait_gather(s)
                start_write(s, g0 + s)
            for s in range(nbuf):
                wait_write(s)
                start_gather(s, g0 + nbuf + s)

        g0 = (n_grp - 1) * nbuf
        for s in range(nbuf):
            wait_gather(s)
            start_write(s, g0 + s)
        for s in range(nbuf):
            wait_write(s)

    return k(table, idx)


def _sc_gather_add(ta, tb, ia, ib, dim, ch=64, nbuf=4):
    n = ia.shape[0]
    per_w = n // NW
    n_ch = per_w // ch
    n_grp = n_ch // nbuf
    assert per_w % ch == 0 and n_ch % nbuf == 0
    has_ib = ib is not None
    spmem = ta.shape[0] * dim * 8 <= 6 * 2**20
    mesh = plsc.VectorSubcoreMesh(
        core_axis_name="c", subcore_axis_name="s", num_cores=NC, num_subcores=NS
    )

    @functools.partial(
        pl.kernel,
        out_type=jax.ShapeDtypeStruct((n, dim), jnp.float32),
        mesh=mesh,
        scratch_types=[
            [pltpu.VMEM((per_w,), jnp.int32)] * (2 if has_ib else 1),
            [[pltpu.VMEM((ch, dim), jnp.float32)] * nbuf] * 3,
            [pltpu.SemaphoreType.DMA] * nbuf,
            [pltpu.SemaphoreType.DMA] * nbuf,
            [pltpu.VMEM_SHARED(ta.shape, jnp.float32)] * (2 if spmem else 0),
            [pltpu.SemaphoreType.DMA] * (1 if spmem else 0),
        ],
    )
    def k(ta_hbm, tb_hbm, *rest):
        i_hbm = rest[: (2 if has_ib else 1)]
        o_hbm, idx_v, bufs, sem_g, sem_w, shared, sem_t = rest[2 if has_ib else 1 :]
        bufA, bufB, bufO = bufs
        if spmem:
            @pl.when(lax.axis_index("s") == 0)
            def _():
                pltpu.make_async_copy(ta_hbm, shared[0], sem_t[0]).start()
                pltpu.make_async_copy(tb_hbm, shared[1], sem_t[0]).start()
                pltpu.make_async_copy(ta_hbm, shared[0], sem_t[0]).wait()
                pltpu.make_async_copy(tb_hbm, shared[1], sem_t[0]).wait()

            plsc.subcore_barrier()
            ta_src, tb_src = shared[0], shared[1]
        else:
            ta_src, tb_src = ta_hbm, tb_hbm
        wid = lax.axis_index("s") * NC + lax.axis_index("c")
        w_base = wid * per_w

        for t in range(2 if has_ib else 1):
            pltpu.sync_copy(i_hbm[t].at[pl.ds(w_base, per_w)], idx_v[t])

        def _src_b(j):
            if has_ib:
                return tb_src.at[idx_v[1].at[pl.ds(j * ch, ch)]]
            return tb_src.at[pl.ds(w_base + j * ch, ch)]

        def start_gather(s, j):
            pltpu.make_async_copy(
                ta_src.at[idx_v[0].at[pl.ds(j * ch, ch)]], bufA[s], sem_g[s]
            ).start()
            pltpu.make_async_copy(_src_b(j), bufB[s], sem_g[s]).start()

        def wait_gather(s):
            pltpu.make_async_copy(
                ta_src.at[idx_v[0].at[pl.ds(0, ch)]], bufA[s], sem_g[s]
            ).wait()
            pltpu.make_async_copy(_src_b(0), bufB[s], sem_g[s]).wait()

        def start_write(s, j):
            pltpu.make_async_copy(
                bufO[s], o_hbm.at[pl.ds(w_base + j * ch, ch)], sem_w[s]
            ).start()

        def wait_write(s):
            pltpu.make_async_copy(bufO[s], o_hbm.at[pl.ds(0, ch)], sem_w[s]).wait()

        def add(s):
            @pl.loop(0, ch)
            def _(r):
                for kk in range(dim // 16):
                    sl = pl.ds(kk * 16, 16)
                    bufO[s][r, sl] = bufA[s][r, sl] + bufB[s][r, sl]

        for s in range(nbuf):
            start_gather(s, s)

        @pl.loop(0, n_grp)
        def _(grp):
            g0 = grp * nbuf
            for s in range(nbuf):
                wait_gather(s)

                @pl.when(grp > 0)
                def _():
                    wait_write(s)

                add(s)
                start_write(s, g0 + s)

                @pl.when(grp < n_grp - 1)
                def _():
                    start_gather(s, g0 + nbuf + s)

        for s in range(nbuf):
            wait_write(s)

    return k(ta, tb, ia, ib) if has_ib else k(ta, tb, ia)


def _tc_add(a, b, blk=2048):
    n, dim = a.shape

    def body(a_ref, b_ref, o_ref):
        o_ref[...] = a_ref[...] + b_ref[...]

    spec = pl.BlockSpec((blk, dim), lambda i: (i, 0))
    return pl.pallas_call(
        body,
        grid=(n // blk,),
        in_specs=[spec, spec],
        out_specs=spec,
        out_shape=jax.ShapeDtypeStruct((n, dim), jnp.float32),
    )(a, b)


def kernel(seqs, d, c, u, Wd, Wc, Wu):
    B, L, U = seqs.shape
    BL = B * L
    dd = Wd.shape[1]

    def t_flat(i2d):
        return i2d.reshape(B, L).astype(jnp.int32).T.reshape(BL)

    u_idx = t_flat(u)
    d_idx = t_flat(d)
    c_idx = t_flat(c)
    WdP = jnp.pad(Wd, ((0, 0), (0, U - dd)))
    WcP = jnp.pad(Wc, ((0, 0), (dd, 0)))

    dc_idx = jnp.stack([d_idx, c_idx + Wd.shape[0]], axis=-1).reshape(2 * BL)
    Wdc = jnp.concatenate([Wd, Wc], axis=0)
    seqs_t = seqs.transpose(1, 0, 2).reshape(BL, U)
    seqs_u = _sc_gather_add(Wu, seqs_t, u_idx, None, U, ch=128, nbuf=2)
    d_c = _sc_gather(Wdc, dc_idx, dd, tc_tiling=False).reshape(BL, U)

    def untranspose(flat):
        return flat.reshape(L, B, U).transpose(1, 0, 2)

    return untranspose(seqs_u), untranspose(d_c)


# speedup vs baseline: 1.2180x; 1.2180x over previous
import functools

import jax
import jax.numpy as jnp
from jax import lax
from jax.experimental import pallas as pl
from jax.experimental.pallas import tpu as pltpu
from jax.experimental.pallas import tpu_sc as plsc

NC, NS = 2, 16
NW = NC * NS


def _sc_gather(table, idx, dim, ch=128, nbuf=5):
    n = idx.shape[0]
    per_w = n // NW
    n_ch = per_w // ch
    n_grp = n_ch // nbuf
    assert per_w % ch == 0 and n_ch % nbuf == 0
    mesh = plsc.VectorSubcoreMesh(
        core_axis_name="c", subcore_axis_name="s", num_cores=NC, num_subcores=NS
    )

    @functools.partial(
        pl.kernel,
        out_type=jax.ShapeDtypeStruct((n, dim), jnp.float32),
        mesh=mesh,
        scratch_types=[
            pltpu.VMEM((per_w,), jnp.int32),
            [pltpu.VMEM((ch, dim), jnp.float32)] * nbuf,
            [pltpu.SemaphoreType.DMA] * nbuf,
            [pltpu.SemaphoreType.DMA] * nbuf,
        ],
    )
    def k(table_hbm, idx_hbm, out_hbm, idx_v, bufs, sem_g, sem_w):
        wid = lax.axis_index("s") * NC + lax.axis_index("c")
        w_base = wid * per_w

        pltpu.sync_copy(idx_hbm.at[pl.ds(w_base, per_w)], idx_v)

        def start_gather(s, j):
            pltpu.make_async_copy(
                table_hbm.at[idx_v.at[pl.ds(j * ch, ch)]], bufs[s], sem_g[s]
            ).start()

        def start_write(s, j):
            pltpu.make_async_copy(
                bufs[s], out_hbm.at[pl.ds(w_base + j * ch, ch)], sem_w[s]
            ).start()

        def wait_gather(s):
            pltpu.make_async_copy(
                table_hbm.at[idx_v.at[pl.ds(0, ch)]], bufs[s], sem_g[s]
            ).wait()

        def wait_write(s):
            pltpu.make_async_copy(bufs[s], out_hbm.at[pl.ds(0, ch)], sem_w[s]).wait()

        for s in range(nbuf):
            start_gather(s, s)

        @pl.loop(0, n_grp - 1)
        def _(grp):
            g0 = grp * nbuf
            for s in range(nbuf):
                wait_gather(s)
                start_write(s, g0 + s)
            for s in range(nbuf):
                wait_write(s)
                start_gather(s, g0 + nbuf + s)

        g0 = (n_grp - 1) * nbuf
        for s in range(nbuf):
            wait_gather(s)
            start_write(s, g0 + s)
        for s in range(nbuf):
            wait_write(s)

    return k(table, idx)


def _sc_gather_add(ta, tb, ia, ib, dim, ch=64, nbuf=4):
    n = ia.shape[0]
    per_w = n // NW
    n_ch = per_w // ch
    n_grp = n_ch // nbuf
    assert per_w % ch == 0 and n_ch % nbuf == 0
    has_ib = ib is not None
    spmem = ta.shape[0] * dim * 8 <= 6 * 2**20
    mesh = plsc.VectorSubcoreMesh(
        core_axis_name="c", subcore_axis_name="s", num_cores=NC, num_subcores=NS
    )

    @functools.partial(
        pl.kernel,
        out_type=jax.ShapeDtypeStruct((n, dim), jnp.float32),
        mesh=mesh,
        scratch_types=[
            [pltpu.VMEM((per_w,), jnp.int32)] * (2 if has_ib else 1),
            [[pltpu.VMEM((ch, dim), jnp.float32)] * nbuf] * 3,
            [pltpu.SemaphoreType.DMA] * nbuf,
            [pltpu.SemaphoreType.DMA] * nbuf,
            [pltpu.VMEM_SHARED(ta.shape, jnp.float32)] * (2 if spmem else 0),
            [pltpu.SemaphoreType.DMA] * (1 if spmem else 0),
        ],
    )
    def k(ta_hbm, tb_hbm, *rest):
        i_hbm = rest[: (2 if has_ib else 1)]
        o_hbm, idx_v, bufs, sem_g, sem_w, shared, sem_t = rest[2 if has_ib else 1 :]
        bufA, bufB, bufO = bufs
        if spmem:
            @pl.when(lax.axis_index("s") == 0)
            def _():
                pltpu.make_async_copy(ta_hbm, shared[0], sem_t[0]).start()
                pltpu.make_async_copy(tb_hbm, shared[1], sem_t[0]).start()
                pltpu.make_async_copy(ta_hbm, shared[0], sem_t[0]).wait()
                pltpu.make_async_copy(tb_hbm, shared[1], sem_t[0]).wait()

            plsc.subcore_barrier()
            ta_src, tb_src = shared[0], shared[1]
        else:
            ta_src, tb_src = ta_hbm, tb_hbm
        wid = lax.axis_index("s") * NC + lax.axis_index("c")
        w_base = wid * per_w

        for t in range(2 if has_ib else 1):
            pltpu.sync_copy(i_hbm[t].at[pl.ds(w_base, per_w)], idx_v[t])

        def _src_b(j):
            if has_ib:
                return tb_src.at[idx_v[1].at[pl.ds(j * ch, ch)]]
            return tb_src.at[pl.ds(w_base + j * ch, ch)]

        def start_gather(s, j):
            pltpu.make_async_copy(
                ta_src.at[idx_v[0].at[pl.ds(j * ch, ch)]], bufA[s], sem_g[s]
            ).start()
            pltpu.make_async_copy(_src_b(j), bufB[s], sem_g[s]).start()

        def wait_gather(s):
            pltpu.make_async_copy(
                ta_src.at[idx_v[0].at[pl.ds(0, ch)]], bufA[s], sem_g[s]
            ).wait()
            pltpu.make_async_copy(_src_b(0), bufB[s], sem_g[s]).wait()

        def start_write(s, j):
            pltpu.make_async_copy(
                bufO[s], o_hbm.at[pl.ds(w_base + j * ch, ch)], sem_w[s]
            ).start()

        def wait_write(s):
            pltpu.make_async_copy(bufO[s], o_hbm.at[pl.ds(0, ch)], sem_w[s]).wait()

        def add(s):
            @pl.loop(0, ch)
            def _(r):
                for kk in range(dim // 16):
                    sl = pl.ds(kk * 16, 16)
                    bufO[s][r, sl] = bufA[s][r, sl] + bufB[s][r, sl]

        for s in range(nbuf):
            start_gather(s, s)

        @pl.loop(0, n_grp)
        def _(grp):
            g0 = grp * nbuf
            for s in range(nbuf):
                wait_gather(s)

                @pl.when(grp > 0)
                def _():
                    wait_write(s)

                add(s)
                start_write(s, g0 + s)

                @pl.when(grp < n_grp - 1)
                def _():
                    start_gather(s, g0 + nbuf + s)

        for s in range(nbuf):
            wait_write(s)

    return k(ta, tb, ia, ib) if has_ib else k(ta, tb, ia)


def _tc_add(a, b, blk=2048):
    n, dim = a.shape

    def body(a_ref, b_ref, o_ref):
        o_ref[...] = a_ref[...] + b_ref[...]

    spec = pl.BlockSpec((blk, dim), lambda i: (i, 0))
    return pl.pallas_call(
        body,
        grid=(n // blk,),
        in_specs=[spec, spec],
        out_specs=spec,
        out_shape=jax.ShapeDtypeStruct((n, dim), jnp.float32),
    )(a, b)


def kernel(seqs, d, c, u, Wd, Wc, Wu):
    B, L, U = seqs.shape
    BL = B * L
    dd = Wd.shape[1]

    def t_flat(i2d):
        return i2d.reshape(B, L).astype(jnp.int32).T.reshape(BL)

    u_idx = t_flat(u)
    d_idx = t_flat(d)
    c_idx = t_flat(c)
    WdP = jnp.pad(Wd, ((0, 0), (0, U - dd)))
    WcP = jnp.pad(Wc, ((0, 0), (dd, 0)))

    seqs_t = seqs.transpose(1, 0, 2).reshape(BL, U)
    seqs_u = _sc_gather_add(Wu, seqs_t, u_idx, None, U, ch=128, nbuf=2)
    d_c = _sc_gather_add(WdP, WcP, d_idx, c_idx, U)

    def untranspose(flat):
        return flat.reshape(L, B, U).transpose(1, 0, 2)

    return untranspose(seqs_u), untranspose(d_c)


# speedup vs baseline: 1.2239x; 1.0048x over previous
import functools

import jax
import jax.numpy as jnp
from jax import lax
from jax.experimental import pallas as pl
from jax.experimental.pallas import tpu as pltpu
from jax.experimental.pallas import tpu_sc as plsc

NC, NS = 2, 16
NW = NC * NS


def _sc_gather_add(ta, tb, ia, ib, dim, ch=64, nbuf=4):
    n = ia.shape[0]
    per_w = n // NW
    n_ch = per_w // ch
    n_grp = n_ch // nbuf
    assert per_w % ch == 0 and n_ch % nbuf == 0
    has_ib = ib is not None
    spmem = ta.shape[0] * dim * 8 <= 6 * 2**20
    mesh = plsc.VectorSubcoreMesh(
        core_axis_name="c", subcore_axis_name="s", num_cores=NC, num_subcores=NS
    )

    @functools.partial(
        pl.kernel,
        out_type=jax.ShapeDtypeStruct((n, dim), jnp.float32),
        mesh=mesh,
        scratch_types=[
            [pltpu.VMEM((per_w,), jnp.int32)] * (2 if has_ib else 1),
            [[pltpu.VMEM((ch, dim), jnp.float32)] * nbuf] * 3,
            [pltpu.SemaphoreType.DMA] * nbuf,
            [pltpu.SemaphoreType.DMA] * nbuf,
            [pltpu.VMEM_SHARED(ta.shape, jnp.float32)] * (2 if spmem else 0),
            [pltpu.SemaphoreType.DMA] * (1 if spmem else 0),
        ],
    )
    def k(ta_hbm, tb_hbm, *rest):
        i_hbm = rest[: (2 if has_ib else 1)]
        o_hbm, idx_v, bufs, sem_g, sem_w, shared, sem_t = rest[2 if has_ib else 1 :]
        bufA, bufB, bufO = bufs
        if spmem:
            @pl.when(lax.axis_index("s") == 0)
            def _():
                pltpu.make_async_copy(ta_hbm, shared[0], sem_t[0]).start()
                pltpu.make_async_copy(tb_hbm, shared[1], sem_t[0]).start()
                pltpu.make_async_copy(ta_hbm, shared[0], sem_t[0]).wait()
                pltpu.make_async_copy(tb_hbm, shared[1], sem_t[0]).wait()

            plsc.subcore_barrier()
            ta_src, tb_src = shared[0], shared[1]
        else:
            ta_src, tb_src = ta_hbm, tb_hbm
        wid = lax.axis_index("s") * NC + lax.axis_index("c")
        w_base = wid * per_w

        for t in range(2 if has_ib else 1):
            pltpu.sync_copy(i_hbm[t].at[pl.ds(w_base, per_w)], idx_v[t])

        def _src_b(j):
            if has_ib:
                return tb_src.at[idx_v[1].at[pl.ds(j * ch, ch)]]
            return tb_src.at[pl.ds(w_base + j * ch, ch)]

        def start_gather(s, j):
            pltpu.make_async_copy(
                ta_src.at[idx_v[0].at[pl.ds(j * ch, ch)]], bufA[s], sem_g[s]
            ).start()
            pltpu.make_async_copy(_src_b(j), bufB[s], sem_g[s]).start()

        def wait_gather(s):
            pltpu.make_async_copy(
                ta_src.at[idx_v[0].at[pl.ds(0, ch)]], bufA[s], sem_g[s]
            ).wait()
            pltpu.make_async_copy(_src_b(0), bufB[s], sem_g[s]).wait()

        def start_write(s, j):
            pltpu.make_async_copy(
                bufO[s], o_hbm.at[pl.ds(w_base + j * ch, ch)], sem_w[s]
            ).start()

        def wait_write(s):
            pltpu.make_async_copy(bufO[s], o_hbm.at[pl.ds(0, ch)], sem_w[s]).wait()

        def add(s):
            @pl.loop(0, ch)
            def _(r):
                for kk in range(dim // 16):
                    sl = pl.ds(kk * 16, 16)
                    bufO[s][r, sl] = bufA[s][r, sl] + bufB[s][r, sl]

        for s in range(nbuf):
            start_gather(s, s)

        @pl.loop(0, n_grp)
        def _(grp):
            g0 = grp * nbuf
            for s in range(nbuf):
                wait_gather(s)

                @pl.when(grp > 0)
                def _():
                    wait_write(s)

                add(s)
                start_write(s, g0 + s)

                @pl.when(grp < n_grp - 1)
                def _():
                    start_gather(s, g0 + nbuf + s)

        for s in range(nbuf):
            wait_write(s)

    return k(ta, tb, ia, ib) if has_ib else k(ta, tb, ia)


def kernel(seqs, d, c, u, Wd, Wc, Wu):
    B, L, U = seqs.shape
    BL = B * L
    dd = Wd.shape[1]

    def t_flat(i2d):
        return i2d.reshape(B, L).astype(jnp.int32).T.reshape(BL)

    u_idx = t_flat(u)
    d_idx = t_flat(d)
    c_idx = t_flat(c)
    WdP = jnp.pad(Wd, ((0, 0), (0, U - dd)))
    WcP = jnp.pad(Wc, ((0, 0), (dd, 0)))

    seqs_t = seqs.transpose(1, 0, 2).reshape(BL, U)
    seqs_u = _sc_gather_add(Wu, seqs_t, u_idx, None, U, ch=128, nbuf=2)
    d_c = _sc_gather_add(WdP, WcP, d_idx, c_idx, U)

    def untranspose(flat):
        return flat.reshape(L, B, U).transpose(1, 0, 2)

    return untranspose(seqs_u), untranspose(d_c)
